# baseline (speedup 1.0000x reference)
import functools

import jax
import jax.numpy as jnp
from jax import lax
from jax.experimental import pallas as pl
from jax.experimental.pallas import tpu as pltpu
from jax.experimental.pallas import tpu_sc as plsc

BATCH = 16384
D = 64
NC = 2
NS = 16
NW = NC * NS
BPW = BATCH // NW
NCH = 4
CH = BPW // NCH
NGRP = BPW // 16

def _newton_sqrt(x):
    y = plsc.bitcast(0x5F3759DF - (plsc.bitcast(x, jnp.int32) >> 1), jnp.float32)
    half_x = 0.5 * x
    for _ in range(3):
        y = y * (1.5 - half_x * y * y)
    return x * y


def _sc_kernel(h_hbm, r_hbm, t_hbm, ent_hbm, rel_hbm, out_hbm,
               idx_h, idx_r, idx_t, rows_h, rows_r, rows_t, out_v, sem):
    wid = lax.axis_index("s") * NC + lax.axis_index("c")

    pltpu.sync_copy(h_hbm.at[wid], idx_h)
    pltpu.sync_copy(r_hbm.at[wid], idx_r)
    pltpu.sync_copy(t_hbm.at[wid], idx_t)

    copies = []
    for j in range(NCH):
        sl = pl.ds(j * CH, CH)
        copies.append(pltpu.async_copy(ent_hbm.at[idx_h.at[j]], rows_h.at[sl], sem))
        copies.append(pltpu.async_copy(rel_hbm.at[idx_r.at[j]], rows_r.at[sl], sem))
        copies.append(pltpu.async_copy(ent_hbm.at[idx_t.at[j]], rows_t.at[sl], sem))
    for c in copies:
        c.wait()

    lane = lax.iota(jnp.int32, 16)

    def body(g, carry):
        res = jnp.zeros((16,), jnp.float32)
        for u in range(16):
            b = g * 16 + u
            acc = jnp.zeros((16,), jnp.float32)
            for k in range(D // 16):
                sl = pl.ds(k * 16, 16)
                diff = rows_h[b, sl] + rows_r[b, sl] - rows_t[b, sl]
                acc = acc + diff * diff
            res = jnp.where(lane == u, jnp.sum(acc), res)
        out_v[pl.ds(g * 16, 16)] = _newton_sqrt(res)
        return carry

    lax.fori_loop(0, NGRP, body, 0)

    pltpu.sync_copy(out_v, out_hbm.at[wid])


@jax.jit
def kernel(h, r, t, ent_emb, rel_emb):
    h3 = h.astype(jnp.int32).reshape(NW, NCH, CH)
    r3 = r.astype(jnp.int32).reshape(NW, NCH, CH)
    t3 = t.astype(jnp.int32).reshape(NW, NCH, CH)

    call = functools.partial(
        pl.kernel,
        out_type=jax.ShapeDtypeStruct((NW, BPW), jnp.float32),
        mesh=plsc.VectorSubcoreMesh(core_axis_name="c", subcore_axis_name="s"),
        compiler_params=pltpu.CompilerParams(
            needs_layout_passes=False, use_tc_tiling_on_sc=False),
        scratch_types=[
            pltpu.VMEM((NCH, CH), jnp.int32),
            pltpu.VMEM((NCH, CH), jnp.int32),
            pltpu.VMEM((NCH, CH), jnp.int32),
            pltpu.VMEM((BPW, D), jnp.float32),
            pltpu.VMEM((BPW, D), jnp.float32),
            pltpu.VMEM((BPW, D), jnp.float32),
            pltpu.VMEM((BPW,), jnp.float32),
            pltpu.SemaphoreType.DMA,
        ],
    )(_sc_kernel)
    out = call(h3, r3, t3, ent_emb, rel_emb)
    return out.reshape(BATCH)
